# baseline (speedup 1.0000x reference)
import functools

import jax
import jax.numpy as jnp
from jax import lax
from jax.experimental import pallas as pl
from jax.experimental.pallas import tpu as pltpu
from jax.experimental.pallas import tpu_sc as plsc

VOCAB = 100000
EMBED_DIM = 128
CONTEXT = 20
NUM_NEURONS = 128
FAN_IN = CONTEXT * EMBED_DIM

R_SC = 48640
C_TC = VOCAB - R_SC
K_TC = 6
V_BLK = C_TC // K_TC
N_TILES = 32
PT = R_SC // N_TILES
NCH = 5
CH = PT // NCH


@functools.partial(
    pl.kernel,
    out_type=jax.ShapeDtypeStruct((CONTEXT, EMBED_DIM), jnp.float32),
    mesh=plsc.VectorSubcoreMesh(core_axis_name="c", subcore_axis_name="s"),
    scratch_types=[
        pltpu.VMEM((CONTEXT,), jnp.int32),
        pltpu.VMEM((CONTEXT, EMBED_DIM), jnp.float32),
        pltpu.SemaphoreType.DMA,
    ],
)
def _sc_gather(idx_hbm, table_hbm, out_hbm, idx_v, rows_v, sem):
    wid = lax.axis_index("s") * 2 + lax.axis_index("c")

    @pl.when(wid == 0)
    def _():
        pltpu.sync_copy(idx_hbm, idx_v)
        pltpu.async_copy(table_hbm.at[idx_v], rows_v, sem).wait()
        pltpu.sync_copy(rows_v, out_hbm)


@functools.partial(
    pl.kernel,
    out_type=jax.ShapeDtypeStruct((R_SC,), jnp.float32),
    mesh=plsc.VectorSubcoreMesh(core_axis_name="c", subcore_axis_name="s"),
    scratch_types=[
        pltpu.VMEM((FAN_IN,), jnp.float32),
        pltpu.VMEM((8 * FAN_IN,), jnp.float32),
        pltpu.VMEM((16,), jnp.float32),
        pltpu.VMEM((16,), jnp.float32),
        pltpu.VMEM((NUM_NEURONS,), jnp.float32),
        pltpu.VMEM_SHARED((NUM_NEURONS,), jnp.float32),
        pltpu.VMEM((PT,), jnp.float32),
        pltpu.VMEM((PT,), jnp.float32),
        pltpu.VMEM((CH * 128,), jnp.float32),
        pltpu.VMEM((CH * 128,), jnp.float32),
        pltpu.SemaphoreType.DMA,
        pltpu.SemaphoreType.DMA,
    ],
)
def _sc_matvec(e_hbm, w1_hbm, b1_hbm, w2_hbm, b2_hbm, out_hbm,
               e_v, w1_v, b1w_v, h8_v, h_v, hsh_v, b2_v, out_v,
               bufa, bufb, sem_a, sem_b):
    c = lax.axis_index("c")
    s = lax.axis_index("s")
    wid = c * 16 + s
    base_row = C_TC + wid * PT

    pltpu.sync_copy(e_hbm, e_v)
    pltpu.sync_copy(w1_hbm.at[pl.ds(s * (8 * FAN_IN), 8 * FAN_IN)], w1_v)
    pltpu.sync_copy(b1_hbm.at[pl.ds(8 * s, 16)], b1w_v)
    lane = lax.broadcasted_iota(jnp.int32, (16,), 0)

    _gdn = lax.GatherDimensionNumbers(
        offset_dims=(), collapsed_slice_dims=(0,), start_index_map=(0,))

    def lanesum(t):
        for k in (8, 4, 2, 1):
            t = t + lax.gather(
                t, (lane ^ k).reshape(16, 1), _gdn, (1,),
                mode=lax.GatherScatterMode.PROMISE_IN_BOUNDS)
        return t

    hreg = jnp.zeros((16,), jnp.float32)
    for n in range(8):
        def hbody(j, acc, n=n):
            return acc + (e_v[pl.ds(j * 16, 16)]
                          * w1_v[pl.ds(n * FAN_IN + j * 16, 16)])
        acc = lax.fori_loop(0, FAN_IN // 16, hbody,
                            jnp.zeros((16,), jnp.float32))
        hreg = jnp.where(lane == n, lanesum(acc), hreg)
    h8_v[...] = jnp.maximum(hreg + b1w_v[...], 0.0)
    pltpu.sync_copy(h8_v.at[pl.ds(0, 8)], hsh_v.at[pl.ds(8 * s, 8)])
    plsc.subcore_barrier()
    pltpu.sync_copy(hsh_v, h_v)

    pltpu.sync_copy(b2_hbm.at[pl.ds(base_row, PT)], b2_v)
    hregs = [h_v[pl.ds(16 * j, 16)] for j in range(8)]

    bufs = (bufa, bufb)
    sems = (sem_a, sem_b)

    def start(ch):
        return pltpu.async_copy(
            w2_hbm.at[pl.ds((base_row + ch * CH) * 128, CH * 128)],
            bufs[ch % 2], sems[ch % 2])

    lane16 = lax.broadcasted_iota(jnp.int32, (16,), 0)

    def compute(ch):
        buf = bufs[ch % 2]

        def rbody(g, _):
            res = jnp.zeros((16,), jnp.float32)
            for u in range(16):
                off = (g * 16 + u) * 128
                p = [buf[pl.ds(off + 16 * j, 16)] * hregs[j]
                     for j in range(8)]
                t = ((p[0] + p[1]) + (p[2] + p[3])) + \
                    ((p[4] + p[5]) + (p[6] + p[7]))
                res = jnp.where(lane16 == u, lanesum(t), res)
            out_v[pl.ds(ch * CH + g * 16, 16)] = res
            return 0

        lax.fori_loop(0, CH // 16, rbody, 0)

    cps = {0: start(0), 1: start(1)}
    for ch in range(NCH):
        cps[ch].wait()
        compute(ch)
        if ch + 2 < NCH:
            cps[ch + 2] = start(ch + 2)

    def bbody(k, _):
        out_v[pl.ds(k * 16, 16)] = (out_v[pl.ds(k * 16, 16)]
                                    + b2_v[pl.ds(k * 16, 16)])
        return 0
    lax.fori_loop(0, PT // 16, bbody, 0)

    pltpu.sync_copy(out_v, out_hbm.at[pl.ds(wid * PT, PT)])


def _tc_body(e_ref, w1_ref, b1_ref, w2_ref, b2_ref, out_ref, h_ref):
    i = pl.program_id(0)

    @pl.when(i == 0)
    def _init():
        h = lax.dot_general(
            e_ref[...], w1_ref[...],
            (((1,), (1,)), ((), ())),
            preferred_element_type=jnp.float32,
        )
        h_ref[...] = jnp.maximum(h + b1_ref[...], 0.0)

    out_ref[pl.ds(i, 1), :] = lax.dot_general(
        h_ref[...], w2_ref[...],
        (((1,), (1,)), ((), ())),
        preferred_element_type=jnp.float32,
    ) + b2_ref[pl.ds(i, 1), :]


def _combine_body(tc_ref, sc_ref, otc_ref, osc_ref):
    t = tc_ref[...]
    sc = sc_ref[...]
    m = jnp.maximum(jnp.max(t), jnp.max(sc))
    ssum = jnp.sum(jnp.exp(t - m)) + jnp.sum(jnp.exp(sc - m))
    lse = m + jnp.log(ssum)
    otc_ref[...] = t - lse
    osc_ref[...] = sc - lse


def kernel(inputs, emb, W1, b1, W2, b2):
    rows = _sc_gather(inputs, emb)
    e1 = rows.reshape(1, FAN_IN)
    ef = rows.reshape(FAN_IN)
    b1r = b1.reshape(1, NUM_NEURONS)
    b1p = jnp.pad(b1, (0, 16))
    b2tc = b2[:C_TC].reshape(K_TC, V_BLK)

    scl = _sc_matvec(ef, W1.reshape(-1), b1p, W2.reshape(-1), b2)

    tc2d = pl.pallas_call(
        _tc_body,
        grid=(K_TC,),
        in_specs=[
            pl.BlockSpec((1, FAN_IN), lambda i: (0, 0)),
            pl.BlockSpec((NUM_NEURONS, FAN_IN), lambda i: (0, 0)),
            pl.BlockSpec((1, NUM_NEURONS), lambda i: (0, 0)),
            pl.BlockSpec((V_BLK, EMBED_DIM), lambda i: (i, 0)),
            pl.BlockSpec((K_TC, V_BLK), lambda i: (0, 0)),
        ],
        out_specs=pl.BlockSpec((K_TC, V_BLK), lambda i: (0, 0)),
        out_shape=jax.ShapeDtypeStruct((K_TC, V_BLK), jnp.float32),
        scratch_shapes=[pltpu.VMEM((1, NUM_NEURONS), jnp.float32)],
    )(e1, W1, b1r, W2, b2tc)

    sc2d = scl.reshape(R_SC // 128, 128)

    otc, osc = pl.pallas_call(
        _combine_body,
        out_shape=[
            jax.ShapeDtypeStruct((K_TC, V_BLK), jnp.float32),
            jax.ShapeDtypeStruct((R_SC // 128, 128), jnp.float32),
        ],
    )(tc2d, sc2d)

    return jnp.concatenate(
        [otc.reshape(-1), osc.reshape(-1)]).reshape(1, VOCAB)


# speedup vs baseline: 1.0553x; 1.0553x over previous
import functools

import jax
import jax.numpy as jnp
from jax import lax
from jax.experimental import pallas as pl
from jax.experimental.pallas import tpu as pltpu
from jax.experimental.pallas import tpu_sc as plsc

VOCAB = 100000
EMBED_DIM = 128
CONTEXT = 20
NUM_NEURONS = 128
FAN_IN = CONTEXT * EMBED_DIM

R_SC = 30720
C_TC = VOCAB - R_SC
K_TC = 4
V_BLK = C_TC // K_TC
N_TILES = 32
PT = R_SC // N_TILES
NCH = 5
CH = PT // NCH


@functools.partial(
    pl.kernel,
    out_type=jax.ShapeDtypeStruct((CONTEXT, EMBED_DIM), jnp.float32),
    mesh=plsc.VectorSubcoreMesh(core_axis_name="c", subcore_axis_name="s"),
    scratch_types=[
        pltpu.VMEM((CONTEXT,), jnp.int32),
        pltpu.VMEM((CONTEXT, EMBED_DIM), jnp.float32),
        pltpu.SemaphoreType.DMA,
    ],
)
def _sc_gather(idx_hbm, table_hbm, out_hbm, idx_v, rows_v, sem):
    wid = lax.axis_index("s") * 2 + lax.axis_index("c")

    @pl.when(wid == 0)
    def _():
        pltpu.sync_copy(idx_hbm, idx_v)
        pltpu.async_copy(table_hbm.at[idx_v], rows_v, sem).wait()
        pltpu.sync_copy(rows_v, out_hbm)


@functools.partial(
    pl.kernel,
    out_type=jax.ShapeDtypeStruct((R_SC,), jnp.float32),
    mesh=plsc.VectorSubcoreMesh(core_axis_name="c", subcore_axis_name="s"),
    scratch_types=[
        pltpu.VMEM((FAN_IN,), jnp.float32),
        pltpu.VMEM((8 * FAN_IN,), jnp.float32),
        pltpu.VMEM((16,), jnp.float32),
        pltpu.VMEM((16,), jnp.float32),
        pltpu.VMEM((NUM_NEURONS,), jnp.float32),
        pltpu.VMEM_SHARED((NUM_NEURONS,), jnp.float32),
        pltpu.VMEM((PT,), jnp.float32),
        pltpu.VMEM((PT,), jnp.float32),
        pltpu.VMEM((CH * 128,), jnp.float32),
        pltpu.VMEM((CH * 128,), jnp.float32),
        pltpu.SemaphoreType.DMA,
        pltpu.SemaphoreType.DMA,
    ],
)
def _sc_matvec(e_hbm, w1_hbm, b1_hbm, w2_hbm, b2_hbm, out_hbm,
               e_v, w1_v, b1w_v, h8_v, h_v, hsh_v, b2_v, out_v,
               bufa, bufb, sem_a, sem_b):
    c = lax.axis_index("c")
    s = lax.axis_index("s")
    wid = c * 16 + s
    base_row = C_TC + wid * PT

    pltpu.sync_copy(e_hbm, e_v)
    pltpu.sync_copy(w1_hbm.at[pl.ds(s * (8 * FAN_IN), 8 * FAN_IN)], w1_v)
    pltpu.sync_copy(b1_hbm.at[pl.ds(8 * s, 16)], b1w_v)
    lane = lax.broadcasted_iota(jnp.int32, (16,), 0)

    _gdn = lax.GatherDimensionNumbers(
        offset_dims=(), collapsed_slice_dims=(0,), start_index_map=(0,))

    def lanesum(t):
        for k in (8, 4, 2, 1):
            t = t + lax.gather(
                t, (lane ^ k).reshape(16, 1), _gdn, (1,),
                mode=lax.GatherScatterMode.PROMISE_IN_BOUNDS)
        return t

    hreg = jnp.zeros((16,), jnp.float32)
    for n in range(8):
        def hbody(j, acc, n=n):
            return acc + (e_v[pl.ds(j * 16, 16)]
                          * w1_v[pl.ds(n * FAN_IN + j * 16, 16)])
        acc = lax.fori_loop(0, FAN_IN // 16, hbody,
                            jnp.zeros((16,), jnp.float32))
        hreg = jnp.where(lane == n, lanesum(acc), hreg)
    h8_v[...] = jnp.maximum(hreg + b1w_v[...], 0.0)
    pltpu.sync_copy(h8_v.at[pl.ds(0, 8)], hsh_v.at[pl.ds(8 * s, 8)])
    plsc.subcore_barrier()
    pltpu.sync_copy(hsh_v, h_v)

    pltpu.sync_copy(b2_hbm.at[pl.ds(base_row, PT)], b2_v)
    hregs = [h_v[pl.ds(16 * j, 16)] for j in range(8)]

    bufs = (bufa, bufb)
    sems = (sem_a, sem_b)

    def start(ch):
        return pltpu.async_copy(
            w2_hbm.at[pl.ds((base_row + ch * CH) * 128, CH * 128)],
            bufs[ch % 2], sems[ch % 2])

    lane16 = lax.broadcasted_iota(jnp.int32, (16,), 0)

    def compute(ch):
        buf = bufs[ch % 2]

        def rbody(g, _):
            res = jnp.zeros((16,), jnp.float32)
            for u in range(16):
                off = (g * 16 + u) * 128
                p = [buf[pl.ds(off + 16 * j, 16)] * hregs[j]
                     for j in range(8)]
                t = ((p[0] + p[1]) + (p[2] + p[3])) + \
                    ((p[4] + p[5]) + (p[6] + p[7]))
                res = jnp.where(lane16 == u, lanesum(t), res)
            out_v[pl.ds(ch * CH + g * 16, 16)] = res
            return 0

        lax.fori_loop(0, CH // 16, rbody, 0)

    cps = {0: start(0), 1: start(1)}
    for ch in range(NCH):
        cps[ch].wait()
        compute(ch)
        if ch + 2 < NCH:
            cps[ch + 2] = start(ch + 2)

    def bbody(k, _):
        out_v[pl.ds(k * 16, 16)] = (out_v[pl.ds(k * 16, 16)]
                                    + b2_v[pl.ds(k * 16, 16)])
        return 0
    lax.fori_loop(0, PT // 16, bbody, 0)

    pltpu.sync_copy(out_v, out_hbm.at[pl.ds(wid * PT, PT)])


def _tc_body(e_ref, w1_ref, b1_ref, w2_ref, b2_ref, out_ref, h_ref):
    i = pl.program_id(0)

    @pl.when(i == 0)
    def _init():
        h = lax.dot_general(
            e_ref[...], w1_ref[...],
            (((1,), (1,)), ((), ())),
            preferred_element_type=jnp.float32,
        )
        h_ref[...] = jnp.maximum(h + b1_ref[...], 0.0)

    out_ref[pl.ds(i, 1), :] = lax.dot_general(
        h_ref[...], w2_ref[...],
        (((1,), (1,)), ((), ())),
        preferred_element_type=jnp.float32,
    ) + b2_ref[pl.ds(i, 1), :]


def _combine_body(tc_ref, sc_ref, otc_ref, osc_ref):
    t = tc_ref[...]
    sc = sc_ref[...]
    m = jnp.maximum(jnp.max(t), jnp.max(sc))
    ssum = jnp.sum(jnp.exp(t - m)) + jnp.sum(jnp.exp(sc - m))
    lse = m + jnp.log(ssum)
    otc_ref[...] = t - lse
    osc_ref[...] = sc - lse


def kernel(inputs, emb, W1, b1, W2, b2):
    rows = _sc_gather(inputs, emb)
    e1 = rows.reshape(1, FAN_IN)
    ef = rows.reshape(FAN_IN)
    b1r = b1.reshape(1, NUM_NEURONS)
    b1p = jnp.pad(b1, (0, 16))
    b2tc = b2[:C_TC].reshape(K_TC, V_BLK)

    scl = _sc_matvec(ef, W1.reshape(-1), b1p, W2.reshape(-1), b2)

    tc2d = pl.pallas_call(
        _tc_body,
        grid=(K_TC,),
        in_specs=[
            pl.BlockSpec((1, FAN_IN), lambda i: (0, 0)),
            pl.BlockSpec((NUM_NEURONS, FAN_IN), lambda i: (0, 0)),
            pl.BlockSpec((1, NUM_NEURONS), lambda i: (0, 0)),
            pl.BlockSpec((V_BLK, EMBED_DIM), lambda i: (i, 0)),
            pl.BlockSpec((K_TC, V_BLK), lambda i: (0, 0)),
        ],
        out_specs=pl.BlockSpec((K_TC, V_BLK), lambda i: (0, 0)),
        out_shape=jax.ShapeDtypeStruct((K_TC, V_BLK), jnp.float32),
        scratch_shapes=[pltpu.VMEM((1, NUM_NEURONS), jnp.float32)],
    )(e1, W1, b1r, W2, b2tc)

    sc2d = scl.reshape(R_SC // 128, 128)

    otc, osc = pl.pallas_call(
        _combine_body,
        out_shape=[
            jax.ShapeDtypeStruct((K_TC, V_BLK), jnp.float32),
            jax.ShapeDtypeStruct((R_SC // 128, 128), jnp.float32),
        ],
    )(tc2d, sc2d)

    return jnp.concatenate(
        [otc.reshape(-1), osc.reshape(-1)]).reshape(1, VOCAB)


# speedup vs baseline: 1.2427x; 1.1776x over previous
import functools

import jax
import jax.numpy as jnp
from jax import lax
from jax.experimental import pallas as pl
from jax.experimental.pallas import tpu as pltpu
from jax.experimental.pallas import tpu_sc as plsc

VOCAB = 100000
EMBED_DIM = 128
CONTEXT = 20
NUM_NEURONS = 128
FAN_IN = CONTEXT * EMBED_DIM

R_SC = 20480
C_TC = VOCAB - R_SC
K_TC = 4
V_BLK = C_TC // K_TC
N_TILES = 32
PT = R_SC // N_TILES
NCH = 5
CH = PT // NCH


@functools.partial(
    pl.kernel,
    out_type=jax.ShapeDtypeStruct((R_SC,), jnp.float32),
    mesh=plsc.VectorSubcoreMesh(core_axis_name="c", subcore_axis_name="s"),
    scratch_types=[
        pltpu.VMEM((CONTEXT,), jnp.int32),
        pltpu.VMEM((CONTEXT, EMBED_DIM), jnp.float32),
        pltpu.VMEM((FAN_IN,), jnp.float32),
        pltpu.VMEM((8 * FAN_IN,), jnp.float32),
        pltpu.VMEM((16,), jnp.float32),
        pltpu.VMEM((16,), jnp.float32),
        pltpu.VMEM((NUM_NEURONS,), jnp.float32),
        pltpu.VMEM_SHARED((NUM_NEURONS,), jnp.float32),
        pltpu.VMEM((PT,), jnp.float32),
        pltpu.VMEM((PT,), jnp.float32),
        pltpu.VMEM((CH * 128,), jnp.float32),
        pltpu.VMEM((CH * 128,), jnp.float32),
        pltpu.SemaphoreType.DMA,
        pltpu.SemaphoreType.DMA,
    ],
)
def _sc_matvec(idx_hbm, emb_hbm, w1_hbm, b1_hbm, w2_hbm, b2_hbm, out_hbm,
               idx_v, rows_v, e_v, w1_v, b1w_v, h8_v, h_v, hsh_v, b2_v, out_v,
               bufa, bufb, sem_a, sem_b):
    c = lax.axis_index("c")
    s = lax.axis_index("s")
    wid = c * 16 + s
    base_row = C_TC + wid * PT

    pltpu.sync_copy(idx_hbm, idx_v)
    pltpu.async_copy(emb_hbm.at[idx_v], rows_v, sem_a).wait()
    for r in range(CONTEXT):
        for j in range(EMBED_DIM // 16):
            e_v[pl.ds(r * EMBED_DIM + 16 * j, 16)] = rows_v[r, pl.ds(16 * j, 16)]

    pltpu.sync_copy(w1_hbm.at[pl.ds(s * (8 * FAN_IN), 8 * FAN_IN)], w1_v)
    pltpu.sync_copy(b1_hbm.at[pl.ds(8 * s, 16)], b1w_v)
    lane = lax.broadcasted_iota(jnp.int32, (16,), 0)

    _gdn = lax.GatherDimensionNumbers(
        offset_dims=(), collapsed_slice_dims=(0,), start_index_map=(0,))

    def lanesum(t):
        for k in (8, 4, 2, 1):
            t = t + lax.gather(
                t, (lane ^ k).reshape(16, 1), _gdn, (1,),
                mode=lax.GatherScatterMode.PROMISE_IN_BOUNDS)
        return t

    hreg = jnp.zeros((16,), jnp.float32)
    for n in range(8):
        def hbody(j, acc, n=n):
            return acc + (e_v[pl.ds(j * 16, 16)]
                          * w1_v[pl.ds(n * FAN_IN + j * 16, 16)])
        acc = lax.fori_loop(0, FAN_IN // 16, hbody,
                            jnp.zeros((16,), jnp.float32))
        hreg = jnp.where(lane == n, lanesum(acc), hreg)
    h8_v[...] = jnp.maximum(hreg + b1w_v[...], 0.0)
    pltpu.sync_copy(h8_v.at[pl.ds(0, 8)], hsh_v.at[pl.ds(8 * s, 8)])
    plsc.subcore_barrier()
    pltpu.sync_copy(hsh_v, h_v)

    pltpu.sync_copy(b2_hbm.at[pl.ds(base_row, PT)], b2_v)
    hregs = [h_v[pl.ds(16 * j, 16)] for j in range(8)]

    bufs = (bufa, bufb)
    sems = (sem_a, sem_b)

    def start(ch):
        return pltpu.async_copy(
            w2_hbm.at[pl.ds((base_row + ch * CH) * 128, CH * 128)],
            bufs[ch % 2], sems[ch % 2])

    lane16 = lax.broadcasted_iota(jnp.int32, (16,), 0)

    def compute(ch):
        buf = bufs[ch % 2]

        def rbody(g, _):
            res = jnp.zeros((16,), jnp.float32)
            for u in range(16):
                off = (g * 16 + u) * 128
                p = [buf[pl.ds(off + 16 * j, 16)] * hregs[j]
                     for j in range(8)]
                t = ((p[0] + p[1]) + (p[2] + p[3])) + \
                    ((p[4] + p[5]) + (p[6] + p[7]))
                res = jnp.where(lane16 == u, lanesum(t), res)
            out_v[pl.ds(ch * CH + g * 16, 16)] = res
            return 0

        lax.fori_loop(0, CH // 16, rbody, 0)

    cps = {0: start(0), 1: start(1)}
    for ch in range(NCH):
        cps[ch].wait()
        compute(ch)
        if ch + 2 < NCH:
            cps[ch + 2] = start(ch + 2)

    def bbody(k, _):
        out_v[pl.ds(k * 16, 16)] = (out_v[pl.ds(k * 16, 16)]
                                    + b2_v[pl.ds(k * 16, 16)])
        return 0
    lax.fori_loop(0, PT // 16, bbody, 0)

    pltpu.sync_copy(out_v, out_hbm.at[pl.ds(wid * PT, PT)])


def _tc_body(idx_ref, emb_ref, w1_ref, b1_ref, w2_ref, b2_ref, out_ref,
             rows_v, h_ref, sem):
    i = pl.program_id(0)

    @pl.when(i == 0)
    def _init():
        cps = []
        for r in range(CONTEXT):
            cp = pltpu.make_async_copy(
                emb_ref.at[pl.ds(idx_ref[r], 1), :],
                rows_v.at[pl.ds(r, 1), :],
                sem)
            cp.start()
            cps.append(cp)
        for cp in cps:
            cp.wait()
        h = b1_ref[...]
        for r in range(CONTEXT):
            h = h + lax.dot_general(
                rows_v[pl.ds(r, 1), :],
                w1_ref[:, r * EMBED_DIM:(r + 1) * EMBED_DIM],
                (((1,), (1,)), ((), ())),
                preferred_element_type=jnp.float32,
            )
        h_ref[...] = jnp.maximum(h, 0.0)

    out_ref[pl.ds(i, 1), :] = lax.dot_general(
        h_ref[...], w2_ref[...],
        (((1,), (1,)), ((), ())),
        preferred_element_type=jnp.float32,
    ) + b2_ref[pl.ds(i, 1), :]


def _combine_body(tc_ref, sc_ref, out_ref):
    t = tc_ref[...]
    s = sc_ref[...]
    m = jnp.maximum(jnp.max(t), jnp.max(s))
    ssum = jnp.sum(jnp.exp(t - m)) + jnp.sum(jnp.exp(s - m))
    lse = m + jnp.log(ssum)
    for k in range(K_TC):
        out_ref[0:1, pl.ds(k * V_BLK, V_BLK)] = t[k:k + 1, :] - lse
    out_ref[0:1, pl.ds(C_TC, R_SC)] = s - lse


def kernel(inputs, emb, W1, b1, W2, b2):
    idx = inputs.astype(jnp.int32)
    b1r = b1.reshape(1, NUM_NEURONS)
    b1p = jnp.pad(b1, (0, 16))
    b2tc = b2[:C_TC].reshape(K_TC, V_BLK)

    scl = _sc_matvec(idx, emb, W1.reshape(-1), b1p, W2.reshape(-1), b2)

    tc2d = pl.pallas_call(
        _tc_body,
        grid_spec=pltpu.PrefetchScalarGridSpec(
            num_scalar_prefetch=1,
            grid=(K_TC,),
            in_specs=[
                pl.BlockSpec(memory_space=pltpu.MemorySpace.HBM),
                pl.BlockSpec((NUM_NEURONS, FAN_IN), lambda i, idx: (0, 0)),
                pl.BlockSpec((1, NUM_NEURONS), lambda i, idx: (0, 0)),
                pl.BlockSpec((V_BLK, EMBED_DIM), lambda i, idx: (i, 0)),
                pl.BlockSpec((K_TC, V_BLK), lambda i, idx: (0, 0)),
            ],
            out_specs=pl.BlockSpec((K_TC, V_BLK), lambda i, idx: (0, 0)),
            scratch_shapes=[
                pltpu.VMEM((CONTEXT, EMBED_DIM), jnp.float32),
                pltpu.VMEM((1, NUM_NEURONS), jnp.float32),
                pltpu.SemaphoreType.DMA,
            ],
        ),
        out_shape=jax.ShapeDtypeStruct((K_TC, V_BLK), jnp.float32),
    )(idx, emb, W1, b1r, W2, b2tc)

    return pl.pallas_call(
        _combine_body,
        out_shape=jax.ShapeDtypeStruct((1, VOCAB), jnp.float32),
    )(tc2d, scl.reshape(1, R_SC))


# speedup vs baseline: 1.3315x; 1.0715x over previous
import functools

import jax
import jax.numpy as jnp
from jax import lax
from jax.experimental import pallas as pl
from jax.experimental.pallas import tpu as pltpu
from jax.experimental.pallas import tpu_sc as plsc

VOCAB = 100000
EMBED_DIM = 128
CONTEXT = 20
NUM_NEURONS = 128
FAN_IN = CONTEXT * EMBED_DIM

R_SC = 20480
C_TC = VOCAB - R_SC
K_TC = 10
V_BLK = C_TC // K_TC
N_TILES = 32
PT = R_SC // N_TILES
NCH = 5
CH = PT // NCH


@functools.partial(
    pl.kernel,
    out_type=jax.ShapeDtypeStruct((R_SC,), jnp.float32),
    mesh=plsc.VectorSubcoreMesh(core_axis_name="c", subcore_axis_name="s"),
    scratch_types=[
        pltpu.VMEM((CONTEXT,), jnp.int32),
        pltpu.VMEM((CONTEXT, EMBED_DIM), jnp.float32),
        pltpu.VMEM((FAN_IN,), jnp.float32),
        pltpu.VMEM((8, FAN_IN), jnp.float32),
        pltpu.VMEM((NUM_NEURONS + 16,), jnp.float32),
        pltpu.VMEM((16,), jnp.float32),
        pltpu.VMEM((NUM_NEURONS,), jnp.float32),
        pltpu.VMEM_SHARED((NUM_NEURONS,), jnp.float32),
        pltpu.VMEM((PT,), jnp.float32),
        pltpu.VMEM((PT,), jnp.float32),
        pltpu.VMEM((CH * 128,), jnp.float32),
        pltpu.VMEM((CH * 128,), jnp.float32),
        pltpu.SemaphoreType.DMA,
        pltpu.SemaphoreType.DMA,
    ],
)
def _sc_matvec(idx_hbm, emb_hbm, w1_hbm, b1_hbm, w2_hbm, b2_hbm, out_hbm,
               idx_v, rows_v, e_v, w1_v, b1f_v, h8_v, h_v, hsh_v, b2_v, out_v,
               bufa, bufb, sem_a, sem_b):
    c = lax.axis_index("c")
    s = lax.axis_index("s")
    wid = c * 16 + s
    base_row = C_TC + wid * PT

    pltpu.sync_copy(idx_hbm, idx_v)
    pltpu.async_copy(emb_hbm.at[idx_v], rows_v, sem_a).wait()
    for r in range(CONTEXT):
        for j in range(EMBED_DIM // 16):
            e_v[pl.ds(r * EMBED_DIM + 16 * j, 16)] = rows_v[r, pl.ds(16 * j, 16)]

    pltpu.sync_copy(w1_hbm.at[pl.ds(8 * s, 8), :], w1_v)
    pltpu.sync_copy(b1_hbm, b1f_v.at[pl.ds(0, NUM_NEURONS)])
    lane = lax.broadcasted_iota(jnp.int32, (16,), 0)

    _gdn = lax.GatherDimensionNumbers(
        offset_dims=(), collapsed_slice_dims=(0,), start_index_map=(0,))

    def lanesum(t):
        for k in (8, 4, 2, 1):
            t = t + lax.gather(
                t, (lane ^ k).reshape(16, 1), _gdn, (1,),
                mode=lax.GatherScatterMode.PROMISE_IN_BOUNDS)
        return t

    def hbody(j, accs):
        ech = e_v[pl.ds(16 * j, 16)]
        return tuple(accs[n] + ech * w1_v[n, pl.ds(16 * j, 16)]
                     for n in range(8))
    accs = lax.fori_loop(
        0, FAN_IN // 16, hbody,
        tuple(jnp.zeros((16,), jnp.float32) for _ in range(8)))
    hreg = jnp.zeros((16,), jnp.float32)
    for n in range(8):
        hreg = jnp.where(lane == n, lanesum(accs[n]), hreg)
    b1w = b1f_v[pl.ds(8 * s, 16)]
    h8_v[...] = jnp.maximum(hreg + b1w, 0.0)
    pltpu.sync_copy(h8_v.at[pl.ds(0, 8)], hsh_v.at[pl.ds(8 * s, 8)])
    plsc.subcore_barrier()
    pltpu.sync_copy(hsh_v, h_v)

    pltpu.sync_copy(b2_hbm.at[pl.ds(base_row, PT)], b2_v)
    hregs = [h_v[pl.ds(16 * j, 16)] for j in range(8)]

    bufs = (bufa, bufb)
    sems = (sem_a, sem_b)

    def start(ch):
        return pltpu.async_copy(
            w2_hbm.at[pl.ds((base_row + ch * CH) * 128, CH * 128)],
            bufs[ch % 2], sems[ch % 2])

    lane16 = lax.broadcasted_iota(jnp.int32, (16,), 0)

    def compute(ch):
        buf = bufs[ch % 2]

        def rbody(g, _):
            res = jnp.zeros((16,), jnp.float32)
            for u in range(16):
                off = (g * 16 + u) * 128
                p = [buf[pl.ds(off + 16 * j, 16)] * hregs[j]
                     for j in range(8)]
                t = ((p[0] + p[1]) + (p[2] + p[3])) + \
                    ((p[4] + p[5]) + (p[6] + p[7]))
                res = jnp.where(lane16 == u, lanesum(t), res)
            out_v[pl.ds(ch * CH + g * 16, 16)] = res
            return 0

        lax.fori_loop(0, CH // 16, rbody, 0)

    cps = {0: start(0), 1: start(1)}
    for ch in range(NCH):
        cps[ch].wait()
        compute(ch)
        if ch + 2 < NCH:
            cps[ch + 2] = start(ch + 2)

    def bbody(k, _):
        out_v[pl.ds(k * 16, 16)] = (out_v[pl.ds(k * 16, 16)]
                                    + b2_v[pl.ds(k * 16, 16)])
        return 0
    lax.fori_loop(0, PT // 16, bbody, 0)

    pltpu.sync_copy(out_v, out_hbm.at[pl.ds(wid * PT, PT)])


def _tc_body(idx_ref, emb_ref, w1_ref, b1_ref, w2_ref, b2_ref, out_ref,
             rows_v, h_ref, sem):
    i = pl.program_id(0)

    @pl.when(i == 0)
    def _init():
        cps = []
        for r in range(CONTEXT):
            cp = pltpu.make_async_copy(
                emb_ref.at[pl.ds(idx_ref[r], 1), :],
                rows_v.at[pl.ds(r, 1), :],
                sem)
            cp.start()
            cps.append(cp)
        for cp in cps:
            cp.wait()
        h = b1_ref[...]
        for r in range(CONTEXT):
            h = h + lax.dot_general(
                rows_v[pl.ds(r, 1), :],
                w1_ref[:, r * EMBED_DIM:(r + 1) * EMBED_DIM],
                (((1,), (1,)), ((), ())),
                preferred_element_type=jnp.float32,
            )
        h_ref[...] = jnp.maximum(h, 0.0)

    out_ref[pl.ds(i, 1), :] = lax.dot_general(
        h_ref[...], w2_ref[...],
        (((1,), (1,)), ((), ())),
        preferred_element_type=jnp.float32,
    ) + b2_ref[pl.ds(i, 1), :]


def _combine_body(tc_ref, sc_ref, out_ref):
    t = tc_ref[...]
    s = sc_ref[...]
    m = jnp.maximum(jnp.max(t), jnp.max(s))
    ssum = jnp.sum(jnp.exp(t - m)) + jnp.sum(jnp.exp(s - m))
    lse = m + jnp.log(ssum)
    for k in range(K_TC):
        out_ref[0:1, pl.ds(k * V_BLK, V_BLK)] = t[k:k + 1, :] - lse
    out_ref[0:1, pl.ds(C_TC, R_SC)] = s - lse


def kernel(inputs, emb, W1, b1, W2, b2):
    idx = inputs.astype(jnp.int32)
    b1r = b1.reshape(1, NUM_NEURONS)
    b2tc = b2[:C_TC].reshape(K_TC, V_BLK)

    scl = _sc_matvec(idx, emb, W1, b1, W2.reshape(-1), b2)

    tc2d = pl.pallas_call(
        _tc_body,
        grid_spec=pltpu.PrefetchScalarGridSpec(
            num_scalar_prefetch=1,
            grid=(K_TC,),
            in_specs=[
                pl.BlockSpec(memory_space=pltpu.MemorySpace.HBM),
                pl.BlockSpec((NUM_NEURONS, FAN_IN), lambda i, idx: (0, 0)),
                pl.BlockSpec((1, NUM_NEURONS), lambda i, idx: (0, 0)),
                pl.BlockSpec((V_BLK, EMBED_DIM), lambda i, idx: (i, 0)),
                pl.BlockSpec((K_TC, V_BLK), lambda i, idx: (0, 0)),
            ],
            out_specs=pl.BlockSpec((K_TC, V_BLK), lambda i, idx: (0, 0)),
            scratch_shapes=[
                pltpu.VMEM((CONTEXT, EMBED_DIM), jnp.float32),
                pltpu.VMEM((1, NUM_NEURONS), jnp.float32),
                pltpu.SemaphoreType.DMA,
            ],
        ),
        out_shape=jax.ShapeDtypeStruct((K_TC, V_BLK), jnp.float32),
    )(idx, emb, W1, b1r, W2, b2tc)

    return pl.pallas_call(
        _combine_body,
        out_shape=jax.ShapeDtypeStruct((1, VOCAB), jnp.float32),
    )(tc2d, scl.reshape(1, R_SC))


# speedup vs baseline: 1.3764x; 1.0337x over previous
import functools

import jax
import jax.numpy as jnp
from jax import lax
from jax.experimental import pallas as pl
from jax.experimental.pallas import tpu as pltpu
from jax.experimental.pallas import tpu_sc as plsc

VOCAB = 100000
EMBED_DIM = 128
CONTEXT = 20
NUM_NEURONS = 128
FAN_IN = CONTEXT * EMBED_DIM

R_SC = 16384
C_TC = VOCAB - R_SC
K_TC = 4
V_BLK = 20992
C_PAD = K_TC * V_BLK
N_TILES = 32
PT = R_SC // N_TILES
NCH = 4
CH = PT // NCH


@functools.partial(
    pl.kernel,
    out_type=jax.ShapeDtypeStruct((R_SC,), jnp.float32),
    mesh=plsc.VectorSubcoreMesh(core_axis_name="c", subcore_axis_name="s"),
    scratch_types=[
        pltpu.VMEM((CONTEXT,), jnp.int32),
        pltpu.VMEM((CONTEXT, EMBED_DIM), jnp.float32),
        pltpu.VMEM((FAN_IN,), jnp.float32),
        pltpu.VMEM((8, FAN_IN), jnp.float32),
        pltpu.VMEM((NUM_NEURONS + 16,), jnp.float32),
        pltpu.VMEM((16,), jnp.float32),
        pltpu.VMEM((NUM_NEURONS,), jnp.float32),
        pltpu.VMEM_SHARED((NUM_NEURONS,), jnp.float32),
        pltpu.VMEM((PT,), jnp.float32),
        pltpu.VMEM((PT,), jnp.float32),
        pltpu.VMEM((CH * 128,), jnp.float32),
        pltpu.VMEM((CH * 128,), jnp.float32),
        pltpu.SemaphoreType.DMA,
        pltpu.SemaphoreType.DMA,
    ],
)
def _sc_matvec(idx_hbm, emb_hbm, w1_hbm, b1_hbm, w2_hbm, b2_hbm, out_hbm,
               idx_v, rows_v, e_v, w1_v, b1f_v, h8_v, h_v, hsh_v, b2_v, out_v,
               bufa, bufb, sem_a, sem_b):
    c = lax.axis_index("c")
    s = lax.axis_index("s")
    wid = c * 16 + s
    base_row = C_TC + wid * PT

    pltpu.sync_copy(idx_hbm, idx_v)
    pltpu.async_copy(emb_hbm.at[idx_v], rows_v, sem_a).wait()
    for r in range(CONTEXT):
        for j in range(EMBED_DIM // 16):
            e_v[pl.ds(r * EMBED_DIM + 16 * j, 16)] = rows_v[r, pl.ds(16 * j, 16)]

    pltpu.sync_copy(w1_hbm.at[pl.ds(8 * s, 8), :], w1_v)
    pltpu.sync_copy(b1_hbm, b1f_v.at[pl.ds(0, NUM_NEURONS)])
    lane = lax.broadcasted_iota(jnp.int32, (16,), 0)

    _gdn = lax.GatherDimensionNumbers(
        offset_dims=(), collapsed_slice_dims=(0,), start_index_map=(0,))

    def lanesum(t):
        for k in (8, 4, 2, 1):
            t = t + lax.gather(
                t, (lane ^ k).reshape(16, 1), _gdn, (1,),
                mode=lax.GatherScatterMode.PROMISE_IN_BOUNDS)
        return t

    def hbody(j, accs):
        ech = e_v[pl.ds(16 * j, 16)]
        return tuple(accs[n] + ech * w1_v[n, pl.ds(16 * j, 16)]
                     for n in range(8))
    accs = lax.fori_loop(
        0, FAN_IN // 16, hbody,
        tuple(jnp.zeros((16,), jnp.float32) for _ in range(8)))
    hreg = jnp.zeros((16,), jnp.float32)
    for n in range(8):
        hreg = jnp.where(lane == n, lanesum(accs[n]), hreg)
    b1w = b1f_v[pl.ds(8 * s, 16)]
    h8_v[...] = jnp.maximum(hreg + b1w, 0.0)
    pltpu.sync_copy(h8_v.at[pl.ds(0, 8)], hsh_v.at[pl.ds(8 * s, 8)])
    plsc.subcore_barrier()
    pltpu.sync_copy(hsh_v, h_v)

    pltpu.sync_copy(b2_hbm.at[pl.ds(base_row, PT)], b2_v)
    hregs = [h_v[pl.ds(16 * j, 16)] for j in range(8)]

    bufs = (bufa, bufb)
    sems = (sem_a, sem_b)

    def start(ch):
        return pltpu.async_copy(
            w2_hbm.at[pl.ds((base_row + ch * CH) * 128, CH * 128)],
            bufs[ch % 2], sems[ch % 2])

    lane16 = lax.broadcasted_iota(jnp.int32, (16,), 0)

    def compute(ch):
        buf = bufs[ch % 2]

        def rbody(g, _):
            res = jnp.zeros((16,), jnp.float32)
            for u in range(16):
                off = (g * 16 + u) * 128
                p = [buf[pl.ds(off + 16 * j, 16)] * hregs[j]
                     for j in range(8)]
                t = ((p[0] + p[1]) + (p[2] + p[3])) + \
                    ((p[4] + p[5]) + (p[6] + p[7]))
                res = jnp.where(lane16 == u, lanesum(t), res)
            out_v[pl.ds(ch * CH + g * 16, 16)] = res
            return 0

        lax.fori_loop(0, CH // 16, rbody, 0)

    cps = {0: start(0), 1: start(1)}
    for ch in range(NCH):
        cps[ch].wait()
        compute(ch)
        if ch + 2 < NCH:
            cps[ch + 2] = start(ch + 2)

    def bbody(k, _):
        out_v[pl.ds(k * 16, 16)] = (out_v[pl.ds(k * 16, 16)]
                                    + b2_v[pl.ds(k * 16, 16)])
        return 0
    lax.fori_loop(0, PT // 16, bbody, 0)

    pltpu.sync_copy(out_v, out_hbm.at[pl.ds(wid * PT, PT)])


def _tc_body(idx_ref, emb_ref, w1_ref, b1_ref, w2_ref, b2_ref, out_ref,
             rows_v, h_ref, sem):
    i = pl.program_id(0)

    @pl.when(i == 0)
    def _init():
        cps = []
        for r in range(CONTEXT):
            cp = pltpu.make_async_copy(
                emb_ref.at[pl.ds(idx_ref[r], 1), :],
                rows_v.at[pl.ds(r, 1), :],
                sem)
            cp.start()
            cps.append(cp)
        for cp in cps:
            cp.wait()
        h = b1_ref[...]
        for r in range(CONTEXT):
            h = h + lax.dot_general(
                rows_v[pl.ds(r, 1), :],
                w1_ref[:, r * EMBED_DIM:(r + 1) * EMBED_DIM],
                (((1,), (1,)), ((), ())),
                preferred_element_type=jnp.float32,
            )
        h_ref[...] = jnp.maximum(h, 0.0)

    out_ref[...] = lax.dot_general(
        h_ref[...], w2_ref[...],
        (((1,), (1,)), ((), ())),
        preferred_element_type=jnp.float32,
    ) + b2_ref[...]


def _combine_body(tc_ref, sc_ref, out_ref):
    t = tc_ref[...]
    s = sc_ref[...]
    lane = lax.broadcasted_iota(jnp.int32, (1, C_PAD), 1)
    tm = jnp.where(lane < C_TC, t, -jnp.inf)
    m = jnp.maximum(jnp.max(tm), jnp.max(s))
    ssum = jnp.sum(jnp.exp(tm - m)) + jnp.sum(jnp.exp(s - m))
    lse = m + jnp.log(ssum)
    out_ref[0:1, pl.ds(0, C_TC)] = t[:, :C_TC] - lse
    out_ref[0:1, pl.ds(C_TC, R_SC)] = s - lse


def kernel(inputs, emb, W1, b1, W2, b2):
    idx = inputs.astype(jnp.int32)
    b1r = b1.reshape(1, NUM_NEURONS)
    b2r = b2.reshape(1, VOCAB)

    scl = _sc_matvec(idx, emb, W1, b1, W2.reshape(-1), b2)

    tc2d = pl.pallas_call(
        _tc_body,
        grid_spec=pltpu.PrefetchScalarGridSpec(
            num_scalar_prefetch=1,
            grid=(K_TC,),
            in_specs=[
                pl.BlockSpec(memory_space=pltpu.MemorySpace.HBM),
                pl.BlockSpec((NUM_NEURONS, FAN_IN), lambda i, idx: (0, 0)),
                pl.BlockSpec((1, NUM_NEURONS), lambda i, idx: (0, 0)),
                pl.BlockSpec((V_BLK, EMBED_DIM), lambda i, idx: (i, 0)),
                pl.BlockSpec((1, V_BLK), lambda i, idx: (0, i)),
            ],
            out_specs=pl.BlockSpec((1, V_BLK), lambda i, idx: (0, i)),
            scratch_shapes=[
                pltpu.VMEM((CONTEXT, EMBED_DIM), jnp.float32),
                pltpu.VMEM((1, NUM_NEURONS), jnp.float32),
                pltpu.SemaphoreType.DMA,
            ],
        ),
        out_shape=jax.ShapeDtypeStruct((1, C_PAD), jnp.float32),
    )(idx, emb, W1, b1r, W2, b2r)

    return pl.pallas_call(
        _combine_body,
        out_shape=jax.ShapeDtypeStruct((1, VOCAB), jnp.float32),
    )(tc2d, scl.reshape(1, R_SC))


# speedup vs baseline: 1.4809x; 1.0759x over previous
import functools

import jax
import jax.numpy as jnp
from jax import lax
from jax.experimental import pallas as pl
from jax.experimental.pallas import tpu as pltpu
from jax.experimental.pallas import tpu_sc as plsc

VOCAB = 100000
EMBED_DIM = 128
CONTEXT = 20
NUM_NEURONS = 128

V_BLK = 25600
N_BLOCKS = -(-VOCAB // V_BLK)
V_PAD = N_BLOCKS * V_BLK


@functools.partial(
    pl.kernel,
    out_type=jax.ShapeDtypeStruct((CONTEXT, EMBED_DIM), jnp.float32),
    mesh=plsc.VectorSubcoreMesh(core_axis_name="c", subcore_axis_name="s"),
    scratch_types=[
        pltpu.VMEM((CONTEXT,), jnp.int32),
        pltpu.VMEM((CONTEXT, EMBED_DIM), jnp.float32),
        pltpu.SemaphoreType.DMA,
    ],
)
def _sc_gather(idx_hbm, table_hbm, out_hbm, idx_v, rows_v, sem):
    wid = lax.axis_index("s") * 2 + lax.axis_index("c")

    @pl.when(wid == 0)
    def _():
        pltpu.sync_copy(idx_hbm, idx_v)
        pltpu.async_copy(table_hbm.at[idx_v], rows_v, sem).wait()
        pltpu.sync_copy(rows_v, out_hbm)


def _tc_body(e_ref, w1_ref, b1_ref, w2_ref, b2_ref, out_ref, h_ref, ms_ref):
    i = pl.program_id(0)

    @pl.when(i == 0)
    def _init():
        h = lax.dot_general(
            e_ref[...], w1_ref[...],
            (((1,), (1,)), ((), ())),
            preferred_element_type=jnp.float32,
        )
        h_ref[...] = jnp.maximum(h + b1_ref[...], 0.0)
        ms_ref[0] = -1e30
        ms_ref[1] = 0.0

    blk = lax.dot_general(
        h_ref[...].astype(jnp.bfloat16), w2_ref[...].astype(jnp.bfloat16),
        (((1,), (1,)), ((), ())),
        preferred_element_type=jnp.float32,
    ) + b2_ref[...]

    out_ref[pl.ds(i, 1), :] = blk

    col = i * V_BLK + lax.broadcasted_iota(jnp.int32, (1, V_BLK), 1)
    blk_m = jnp.where(col < VOCAB, blk, -1e30)

    m_old = ms_ref[0]
    s_old = ms_ref[1]
    bm = jnp.max(blk_m)
    m_new = jnp.maximum(m_old, bm)
    s_new = s_old * jnp.exp(m_old - m_new) + jnp.sum(jnp.exp(blk_m - m_new))
    ms_ref[0] = m_new
    ms_ref[1] = s_new

    @pl.when(i == pl.num_programs(0) - 1)
    def _fin():
        lse = ms_ref[0] + jnp.log(ms_ref[1])
        out_ref[...] = out_ref[...] - lse


def kernel(inputs, emb, W1, b1, W2, b2):
    rows = _sc_gather(inputs, emb)
    e = rows.reshape(1, CONTEXT * EMBED_DIM)
    b1r = b1.reshape(1, NUM_NEURONS)
    b2p = jnp.pad(b2, (0, V_PAD - VOCAB)).reshape(1, V_PAD)

    res = pl.pallas_call(
        _tc_body,
        grid=(N_BLOCKS,),
        in_specs=[
            pl.BlockSpec((1, CONTEXT * EMBED_DIM), lambda i: (0, 0)),
            pl.BlockSpec((NUM_NEURONS, CONTEXT * EMBED_DIM), lambda i: (0, 0)),
            pl.BlockSpec((1, NUM_NEURONS), lambda i: (0, 0)),
            pl.BlockSpec((V_BLK, EMBED_DIM), lambda i: (i, 0)),
            pl.BlockSpec((1, V_BLK), lambda i: (0, i)),
        ],
        out_specs=pl.BlockSpec((N_BLOCKS, V_BLK), lambda i: (0, 0)),
        out_shape=jax.ShapeDtypeStruct((N_BLOCKS, V_BLK), jnp.float32),
        scratch_shapes=[
            pltpu.VMEM((1, NUM_NEURONS), jnp.float32),
            pltpu.SMEM((2,), jnp.float32),
        ],
    )(e, W1, b1r, W2, b2p)

    return res.reshape(1, V_PAD)[:, :VOCAB]
